# baseline (speedup 1.0000x reference)
import jax
import jax.numpy as jnp
from jax.experimental import pallas as pl
from jax.experimental.pallas import tpu as pltpu

ROWS = 12
LEN = 65536
SUB = 8
LANES = LEN // SUB
NV = LANES // 128
BINS = 256
EDGES = BINS + 1
DELTA = 255.0 / 256.0
HALF_DELTA = DELTA * 0.5
NACC = 1
EU = 4
EDGES_PAD = 260
RPB = 3


def _edge_sums_kernel(x_ref, out_ref, xh_ref):
    xh_ref[...] = x_ref[0] * 0.5

    def body(i, _):
        e0 = i * EU
        ts = [(e0 + u).astype(jnp.float32) * HALF_DELTA for u in range(EU)]
        accs = [[[jnp.zeros((SUB, 128), jnp.float32) for _ in range(NACC)]
                 for _ in range(RPB)] for _ in range(EU)]
        for v in range(NV):
            for r in range(RPB):
                xv = xh_ref[r, :, v * 128:(v + 1) * 128]
                for u in range(EU):
                    accs[u][r][v % NACC] = (
                        accs[u][r][v % NACC] + jnp.tanh(xv - ts[u]))
        for u in range(EU):
            for r in range(RPB):
                total = accs[u][r][0]
                for a in accs[u][r][1:]:
                    total = total + a
                red = jnp.sum(total, axis=0, keepdims=True)
                out_ref[pl.ds(e0 + u, 1), 0, r, 0, :] = red
        return 0

    jax.lax.fori_loop(0, EDGES_PAD // EU, body, 0)


def _finalize_kernel(p_ref, out_ref):
    q = p_ref[:, 0, 0, :]
    Tc = jnp.dot(q, jnp.ones((128, 8), jnp.float32),
                 preferred_element_type=jnp.float32,
                 precision=jax.lax.Precision.HIGHEST)
    h = Tc[0:BINS, :] - Tc[1:BINS + 1, :]
    s = jnp.dot(jnp.ones((8, BINS), jnp.float32), h,
                preferred_element_type=jnp.float32,
                precision=jax.lax.Precision.HIGHEST)
    out_ref[0] = h * (1.0 / s[0:1, 0:1])


def kernel(x):
    x4 = x.reshape(ROWS // RPB, RPB, SUB, LANES)
    part = pl.pallas_call(
        _edge_sums_kernel,
        grid=(ROWS // RPB,),
        in_specs=[pl.BlockSpec((1, RPB, SUB, LANES), lambda i: (i, 0, 0, 0))],
        out_specs=pl.BlockSpec((EDGES_PAD, 1, RPB, 1, 128),
                               lambda i: (0, i, 0, 0, 0)),
        out_shape=jax.ShapeDtypeStruct((EDGES_PAD, ROWS // RPB, RPB, 1, 128),
                                       jnp.float32),
        scratch_shapes=[pltpu.VMEM((RPB, SUB, LANES), jnp.float32)],
        compiler_params=pltpu.CompilerParams(
            dimension_semantics=("arbitrary",)),
    )(x4)
    part = part.reshape(EDGES_PAD, ROWS, 1, 128)
    out = pl.pallas_call(
        _finalize_kernel,
        grid=(ROWS,),
        in_specs=[pl.BlockSpec((EDGES_PAD, 1, 1, 128), lambda i: (0, i, 0, 0))],
        out_specs=pl.BlockSpec((1, BINS, 8), lambda i: (i, 0, 0)),
        out_shape=jax.ShapeDtypeStruct((ROWS, BINS, 8), jnp.float32),
        compiler_params=pltpu.CompilerParams(
            dimension_semantics=("arbitrary",)),
    )(part)
    return out[:, :, 0].reshape(4, 3, BINS)


# speedup vs baseline: 1.0296x; 1.0296x over previous
import jax
import jax.numpy as jnp
from jax.experimental import pallas as pl
from jax.experimental.pallas import tpu as pltpu

ROWS = 12
LEN = 65536
SUB = 8
LANES = LEN // SUB
NV = LANES // 128
BINS = 256
EDGES = BINS + 1
DELTA = 255.0 / 256.0
HALF_DELTA = DELTA * 0.5
NACC = 1
EU = 2
EDGES_PAD = 258
RPB = 12


def _edge_sums_kernel(x_ref, out_ref, xh_ref):
    xh_ref[...] = x_ref[0] * 0.5

    def body(i, _):
        e0 = i * EU
        ts = [(e0 + u).astype(jnp.float32) * HALF_DELTA for u in range(EU)]
        accs = [[[jnp.zeros((SUB, 128), jnp.float32) for _ in range(NACC)]
                 for _ in range(RPB)] for _ in range(EU)]
        for v in range(NV):
            for r in range(RPB):
                xv = xh_ref[r, :, v * 128:(v + 1) * 128]
                for u in range(EU):
                    accs[u][r][v % NACC] = (
                        accs[u][r][v % NACC] + jnp.tanh(xv - ts[u]))
        for u in range(EU):
            for r in range(RPB):
                total = accs[u][r][0]
                for a in accs[u][r][1:]:
                    total = total + a
                red = jnp.sum(total, axis=0, keepdims=True)
                out_ref[pl.ds(e0 + u, 1), 0, r, 0, :] = red
        return 0

    jax.lax.fori_loop(0, EDGES_PAD // EU, body, 0)


def _finalize_kernel(p_ref, out_ref):
    q = p_ref[:, 0, 0, :]
    Tc = jnp.dot(q, jnp.ones((128, 8), jnp.float32),
                 preferred_element_type=jnp.float32,
                 precision=jax.lax.Precision.HIGHEST)
    h = Tc[0:BINS, :] - Tc[1:BINS + 1, :]
    s = jnp.dot(jnp.ones((8, BINS), jnp.float32), h,
                preferred_element_type=jnp.float32,
                precision=jax.lax.Precision.HIGHEST)
    out_ref[0] = h * (1.0 / s[0:1, 0:1])


def kernel(x):
    x4 = x.reshape(ROWS // RPB, RPB, SUB, LANES)
    part = pl.pallas_call(
        _edge_sums_kernel,
        grid=(ROWS // RPB,),
        in_specs=[pl.BlockSpec((1, RPB, SUB, LANES), lambda i: (i, 0, 0, 0))],
        out_specs=pl.BlockSpec((EDGES_PAD, 1, RPB, 1, 128),
                               lambda i: (0, i, 0, 0, 0)),
        out_shape=jax.ShapeDtypeStruct((EDGES_PAD, ROWS // RPB, RPB, 1, 128),
                                       jnp.float32),
        scratch_shapes=[pltpu.VMEM((RPB, SUB, LANES), jnp.float32)],
        compiler_params=pltpu.CompilerParams(
            dimension_semantics=("arbitrary",)),
    )(x4)
    part = part.reshape(EDGES_PAD, ROWS, 1, 128)
    out = pl.pallas_call(
        _finalize_kernel,
        grid=(ROWS,),
        in_specs=[pl.BlockSpec((EDGES_PAD, 1, 1, 128), lambda i: (0, i, 0, 0))],
        out_specs=pl.BlockSpec((1, BINS, 8), lambda i: (i, 0, 0)),
        out_shape=jax.ShapeDtypeStruct((ROWS, BINS, 8), jnp.float32),
        compiler_params=pltpu.CompilerParams(
            dimension_semantics=("arbitrary",)),
    )(part)
    return out[:, :, 0].reshape(4, 3, BINS)


# speedup vs baseline: 1.0415x; 1.0116x over previous
import jax
import jax.numpy as jnp
from jax.experimental import pallas as pl
from jax.experimental.pallas import tpu as pltpu

ROWS = 12
LEN = 65536
SUB = 8
LANES = LEN // SUB
NV = LANES // 128
BINS = 256
EDGES = BINS + 1
DELTA = 255.0 / 256.0
HALF_DELTA = DELTA * 0.5
NACC = 2
EU = 2
EDGES_PAD = 258
RPB = 6


def _edge_sums_kernel(x_ref, out_ref, xh_ref):
    xh_ref[...] = x_ref[0] * 0.5

    def body(i, _):
        e0 = i * EU
        ts = [(e0 + u).astype(jnp.float32) * HALF_DELTA for u in range(EU)]
        accs = [[[jnp.zeros((SUB, 128), jnp.float32) for _ in range(NACC)]
                 for _ in range(RPB)] for _ in range(EU)]
        for v in range(NV):
            for r in range(RPB):
                xv = xh_ref[r, :, v * 128:(v + 1) * 128]
                for u in range(EU):
                    accs[u][r][v % NACC] = (
                        accs[u][r][v % NACC] + jnp.tanh(xv - ts[u]))
        for u in range(EU):
            for r in range(RPB):
                total = accs[u][r][0]
                for a in accs[u][r][1:]:
                    total = total + a
                red = jnp.sum(total, axis=0, keepdims=True)
                out_ref[pl.ds(e0 + u, 1), 0, r, 0, :] = red
        return 0

    jax.lax.fori_loop(0, EDGES_PAD // EU, body, 0)


def _finalize_kernel(p_ref, out_ref):
    q = p_ref[:, 0, 0, :]
    Tc = jnp.dot(q, jnp.ones((128, 8), jnp.float32),
                 preferred_element_type=jnp.float32,
                 precision=jax.lax.Precision.HIGHEST)
    h = Tc[0:BINS, :] - Tc[1:BINS + 1, :]
    s = jnp.dot(jnp.ones((8, BINS), jnp.float32), h,
                preferred_element_type=jnp.float32,
                precision=jax.lax.Precision.HIGHEST)
    out_ref[0] = h * (1.0 / s[0:1, 0:1])


def kernel(x):
    x4 = x.reshape(ROWS // RPB, RPB, SUB, LANES)
    part = pl.pallas_call(
        _edge_sums_kernel,
        grid=(ROWS // RPB,),
        in_specs=[pl.BlockSpec((1, RPB, SUB, LANES), lambda i: (i, 0, 0, 0))],
        out_specs=pl.BlockSpec((EDGES_PAD, 1, RPB, 1, 128),
                               lambda i: (0, i, 0, 0, 0)),
        out_shape=jax.ShapeDtypeStruct((EDGES_PAD, ROWS // RPB, RPB, 1, 128),
                                       jnp.float32),
        scratch_shapes=[pltpu.VMEM((RPB, SUB, LANES), jnp.float32)],
        compiler_params=pltpu.CompilerParams(
            dimension_semantics=("arbitrary",)),
    )(x4)
    part = part.reshape(EDGES_PAD, ROWS, 1, 128)
    out = pl.pallas_call(
        _finalize_kernel,
        grid=(ROWS,),
        in_specs=[pl.BlockSpec((EDGES_PAD, 1, 1, 128), lambda i: (0, i, 0, 0))],
        out_specs=pl.BlockSpec((1, BINS, 8), lambda i: (i, 0, 0)),
        out_shape=jax.ShapeDtypeStruct((ROWS, BINS, 8), jnp.float32),
        compiler_params=pltpu.CompilerParams(
            dimension_semantics=("arbitrary",)),
    )(part)
    return out[:, :, 0].reshape(4, 3, BINS)


# speedup vs baseline: 1.0514x; 1.0094x over previous
import jax
import jax.numpy as jnp
from jax.experimental import pallas as pl
from jax.experimental.pallas import tpu as pltpu

ROWS = 12
LEN = 65536
SUB = 8
LANES = LEN // SUB
NV = LANES // 128
BINS = 256
EDGES = BINS + 1
DELTA = 255.0 / 256.0
HALF_DELTA = DELTA * 0.5
NACC = 2
EU = 2
EDGES_PAD = 258
RPB = 6


def _edge_sums_kernel(x_ref, out_ref, xh_ref):
    xh_ref[...] = x_ref[0] * 0.5

    def body(i, _):
        e0 = i * EU
        ts = [(e0 + u).astype(jnp.float32) * HALF_DELTA for u in range(EU)]
        accs = [[[jnp.zeros((SUB, 128), jnp.float32) for _ in range(NACC)]
                 for _ in range(RPB)] for _ in range(EU)]
        for v in range(NV):
            for r in range(RPB):
                xv = xh_ref[r, :, v * 128:(v + 1) * 128]
                for u in range(EU):
                    accs[u][r][v % NACC] = (
                        accs[u][r][v % NACC] + jnp.tanh(xv - ts[u]))
        for u in range(EU):
            for r in range(RPB):
                total = accs[u][r][0]
                for a in accs[u][r][1:]:
                    total = total + a
                red = jnp.sum(total, axis=0, keepdims=True)
                out_ref[pl.ds(e0 + u, 1), 0, r, 0, :] = red
        return 0

    jax.lax.fori_loop(0, EDGES_PAD // EU, body, 0)


def _finalize_kernel(p_ref, out_ref):
    q = p_ref[:, 0, 0, :]
    qh = q.astype(jnp.bfloat16)
    ql = (q - qh.astype(jnp.float32)).astype(jnp.bfloat16)
    ones = jnp.ones((128, 8), jnp.bfloat16)
    Tc = (jnp.dot(qh, ones, preferred_element_type=jnp.float32)
          + jnp.dot(ql, ones, preferred_element_type=jnp.float32))
    h = Tc[0:BINS, :] - Tc[1:BINS + 1, :]
    s = Tc[0:1, :] - Tc[BINS:BINS + 1, :]
    out_ref[0] = h * (1.0 / s)


def kernel(x):
    x4 = x.reshape(ROWS // RPB, RPB, SUB, LANES)
    part = pl.pallas_call(
        _edge_sums_kernel,
        grid=(ROWS // RPB,),
        in_specs=[pl.BlockSpec((1, RPB, SUB, LANES), lambda i: (i, 0, 0, 0))],
        out_specs=pl.BlockSpec((EDGES_PAD, 1, RPB, 1, 128),
                               lambda i: (0, i, 0, 0, 0)),
        out_shape=jax.ShapeDtypeStruct((EDGES_PAD, ROWS // RPB, RPB, 1, 128),
                                       jnp.float32),
        scratch_shapes=[pltpu.VMEM((RPB, SUB, LANES), jnp.float32)],
        compiler_params=pltpu.CompilerParams(
            dimension_semantics=("arbitrary",)),
    )(x4)
    part = part.reshape(EDGES_PAD, ROWS, 1, 128)
    out = pl.pallas_call(
        _finalize_kernel,
        grid=(ROWS,),
        in_specs=[pl.BlockSpec((EDGES_PAD, 1, 1, 128), lambda i: (0, i, 0, 0))],
        out_specs=pl.BlockSpec((1, BINS, 8), lambda i: (i, 0, 0)),
        out_shape=jax.ShapeDtypeStruct((ROWS, BINS, 8), jnp.float32),
        compiler_params=pltpu.CompilerParams(
            dimension_semantics=("arbitrary",)),
    )(part)
    return out[:, :, 0].reshape(4, 3, BINS)


# speedup vs baseline: 1.0804x; 1.0276x over previous
import jax
import jax.numpy as jnp
from jax.experimental import pallas as pl
from jax.experimental.pallas import tpu as pltpu

ROWS = 12
LEN = 65536
SUB = 8
LANES = LEN // SUB
NV = LANES // 128
BINS = 256
EDGES = BINS + 1
DELTA = 255.0 / 256.0
HALF_DELTA = DELTA * 0.5
NACC = 2
EU = 2
EDGES_PAD = 258
RPB = 6


def _edge_sums_kernel(x_ref, out_ref, xh_ref):
    xh_ref[...] = x_ref[0] * 0.5

    def body(i, _):
        e0 = i * EU
        ts = [(e0 + u).astype(jnp.float32) * HALF_DELTA for u in range(EU)]
        accs = [[[jnp.zeros((SUB, 128), jnp.float32) for _ in range(NACC)]
                 for _ in range(RPB)] for _ in range(EU)]
        for v in range(NV):
            for r in range(RPB):
                xv = xh_ref[r, :, v * 128:(v + 1) * 128]
                for u in range(EU):
                    accs[u][r][v % NACC] = (
                        accs[u][r][v % NACC] + jnp.tanh(xv - ts[u]))
        for u in range(EU):
            for r in range(RPB):
                total = accs[u][r][0]
                for a in accs[u][r][1:]:
                    total = total + a
                red = jnp.sum(total, axis=0, keepdims=True)
                out_ref[pl.ds(e0 + u, 1), 0, r, 0, :] = red
        return 0

    jax.lax.fori_loop(0, EDGES_PAD // EU, body, 0)


RPF = 6


def _finalize_kernel(p_ref, out_ref):
    ones = jnp.ones((128, 8), jnp.bfloat16)
    for r in range(RPF):
        q = p_ref[:, r, 0, :]
        qh = q.astype(jnp.bfloat16)
        ql = (q - qh.astype(jnp.float32)).astype(jnp.bfloat16)
        Tc = (jnp.dot(qh, ones, preferred_element_type=jnp.float32)
              + jnp.dot(ql, ones, preferred_element_type=jnp.float32))
        h = Tc[0:BINS, :] - Tc[1:BINS + 1, :]
        s = Tc[0:1, :] - Tc[BINS:BINS + 1, :]
        out_ref[r] = h * (1.0 / s)


def kernel(x):
    x4 = x.reshape(ROWS // RPB, RPB, SUB, LANES)
    part = pl.pallas_call(
        _edge_sums_kernel,
        grid=(ROWS // RPB,),
        in_specs=[pl.BlockSpec((1, RPB, SUB, LANES), lambda i: (i, 0, 0, 0))],
        out_specs=pl.BlockSpec((EDGES_PAD, 1, RPB, 1, 128),
                               lambda i: (0, i, 0, 0, 0)),
        out_shape=jax.ShapeDtypeStruct((EDGES_PAD, ROWS // RPB, RPB, 1, 128),
                                       jnp.float32),
        scratch_shapes=[pltpu.VMEM((RPB, SUB, LANES), jnp.float32)],
        compiler_params=pltpu.CompilerParams(
            dimension_semantics=("arbitrary",)),
    )(x4)
    part = part.reshape(EDGES_PAD, ROWS, 1, 128)
    out = pl.pallas_call(
        _finalize_kernel,
        grid=(ROWS // RPF,),
        in_specs=[pl.BlockSpec((EDGES_PAD, RPF, 1, 128),
                               lambda i: (0, i, 0, 0))],
        out_specs=pl.BlockSpec((RPF, BINS, 8), lambda i: (i, 0, 0)),
        out_shape=jax.ShapeDtypeStruct((ROWS, BINS, 8), jnp.float32),
        compiler_params=pltpu.CompilerParams(
            dimension_semantics=("arbitrary",)),
    )(part)
    return out[:, :, 0].reshape(4, 3, BINS)


# speedup vs baseline: 1.0994x; 1.0176x over previous
import jax
import jax.numpy as jnp
from jax.experimental import pallas as pl
from jax.experimental.pallas import tpu as pltpu

ROWS = 12
LEN = 65536
SUB = 8
LANES = LEN // SUB
NV = LANES // 128
BINS = 256
EDGES = BINS + 1
DELTA = 255.0 / 256.0
HALF_DELTA = DELTA * 0.5
NACC = 2
EU = 2
EDGES_PAD = 258
RPB = 6


def _soft_hist_kernel(x_ref, out_ref, xh_ref, ps_ref):
    xh_ref[...] = x_ref[0] * 0.5

    def body(i, _):
        e0 = i * EU
        ts = [(e0 + u).astype(jnp.float32) * HALF_DELTA for u in range(EU)]
        accs = [[[jnp.zeros((SUB, 128), jnp.float32) for _ in range(NACC)]
                 for _ in range(RPB)] for _ in range(EU)]
        for v in range(NV):
            for r in range(RPB):
                xv = xh_ref[r, :, v * 128:(v + 1) * 128]
                for u in range(EU):
                    accs[u][r][v % NACC] = (
                        accs[u][r][v % NACC] + jnp.tanh(xv - ts[u]))
        for u in range(EU):
            for r in range(RPB):
                total = accs[u][r][0]
                for a in accs[u][r][1:]:
                    total = total + a
                red = jnp.sum(total, axis=0, keepdims=True)
                ps_ref[pl.ds(e0 + u, 1), r, 0, :] = red
        return 0

    jax.lax.fori_loop(0, EDGES_PAD // EU, body, 0)

    ones = jnp.ones((128, 8), jnp.bfloat16)
    for r in range(RPB):
        q = ps_ref[:, r, 0, :]
        qh = q.astype(jnp.bfloat16)
        ql = (q - qh.astype(jnp.float32)).astype(jnp.bfloat16)
        Tc = (jnp.dot(qh, ones, preferred_element_type=jnp.float32)
              + jnp.dot(ql, ones, preferred_element_type=jnp.float32))
        h = Tc[0:BINS, :] - Tc[1:BINS + 1, :]
        s = Tc[0:1, :] - Tc[BINS:BINS + 1, :]
        out_ref[r] = h * (1.0 / s)


def kernel(x):
    x4 = x.reshape(ROWS // RPB, RPB, SUB, LANES)
    out = pl.pallas_call(
        _soft_hist_kernel,
        grid=(ROWS // RPB,),
        in_specs=[pl.BlockSpec((1, RPB, SUB, LANES), lambda i: (i, 0, 0, 0))],
        out_specs=pl.BlockSpec((RPB, BINS, 8), lambda i: (i, 0, 0)),
        out_shape=jax.ShapeDtypeStruct((ROWS, BINS, 8), jnp.float32),
        scratch_shapes=[
            pltpu.VMEM((RPB, SUB, LANES), jnp.float32),
            pltpu.VMEM((EDGES_PAD, RPB, 1, 128), jnp.float32),
        ],
        compiler_params=pltpu.CompilerParams(
            dimension_semantics=("arbitrary",)),
    )(x4)
    return out[:, :, 0].reshape(4, 3, BINS)
